# Initial kernel scaffold; baseline (speedup 1.0000x reference)
import jax
import jax.numpy as jnp
from jax.experimental import pallas as pl


def kernel(C_now2past, C_past2now):
    raise NotImplementedError("write your pallas kernel here")


# speedup vs baseline: 15.0156x; 15.0156x over previous
import functools

import jax
import jax.numpy as jnp
import numpy as np
from jax.experimental import pallas as pl

N_COLS = 32768
K_RATIO = 0.1

_POS = np.float32(1.0 - K_RATIO) * np.float32(N_COLS - 1)
_LO_IDX = int(np.floor(_POS))
_GAMMA = float(np.float32(_POS) - np.float32(_LO_IDX))
_RANK = _LO_IDX + 1

_HI_BITS = 0x3F800000
_MAX_BITS = 0x7F7FFFFF


def _row_tau(keys):
    rows = keys.shape[0]
    lo = jnp.zeros((rows, 1), jnp.int32)
    hi = jnp.full((rows, 1), _HI_BITS, jnp.int32)

    def body(_, carry):
        lo, hi = carry
        mid = (lo + hi) >> 1
        cnt = jnp.sum((keys <= mid).astype(jnp.int32), axis=1, keepdims=True)
        take_hi = cnt >= _RANK
        return (jnp.where(take_hi, lo, mid + 1), jnp.where(take_hi, mid, hi))

    lo, hi = jax.lax.fori_loop(0, 30, body, (lo, hi))
    s1_bits = lo

    cnt_leq = jnp.sum((keys <= s1_bits).astype(jnp.int32), axis=1, keepdims=True)
    above = jnp.where(keys > s1_bits, keys, _MAX_BITS)
    nxt_bits = jnp.min(above, axis=1, keepdims=True)
    s2_bits = jnp.where(cnt_leq >= _RANK + 1, s1_bits, nxt_bits)

    s1 = jax.lax.bitcast_convert_type(s1_bits, jnp.float32)
    s2 = jax.lax.bitcast_convert_type(s2_bits, jnp.float32)
    g = jnp.float32(_GAMMA)
    return s1 * (jnp.float32(1) - g) + s2 * g


def _masks_kernel(a_ref, b_ref, new_ref, dis_ref):
    a = a_ref[...]
    b = b_ref[...]
    ka = jax.lax.bitcast_convert_type(a, jnp.int32)
    kb = jax.lax.bitcast_convert_type(b, jnp.int32)
    tau_a = _row_tau(ka)
    tau_b = _row_tau(kb)
    a_hi = a > tau_a
    b_hi = b > tau_b
    new_ref[...] = a_hi & jnp.logical_not(b_hi)
    dis_ref[...] = b_hi & jnp.logical_not(a_hi)


@jax.jit
def kernel(C_now2past, C_past2now):
    rows, cols = C_now2past.shape
    block_rows = 8
    grid = (rows // block_rows,)
    spec = pl.BlockSpec((block_rows, cols), lambda i: (i, 0))
    out_shape = jax.ShapeDtypeStruct((rows, cols), jnp.bool_)
    new_mask, dis_mask = pl.pallas_call(
        _masks_kernel,
        grid=grid,
        in_specs=[spec, spec],
        out_specs=[spec, spec],
        out_shape=[out_shape, out_shape],
    )(C_now2past, C_past2now)
    return (new_mask, dis_mask)


# speedup vs baseline: 25.6103x; 1.7056x over previous
import functools

import jax
import jax.numpy as jnp
import numpy as np
from jax.experimental import pallas as pl

N_COLS = 32768
K_RATIO = 0.1

_POS = np.float32(1.0 - K_RATIO) * np.float32(N_COLS - 1)
_LO_IDX = int(np.floor(_POS))
_GAMMA = float(np.float32(_POS) - np.float32(_LO_IDX))
_RANK = _LO_IDX + 1

_HI_BITS = 0x3F800000
_MAX_BITS = 0x7F7FFFFF


def _row_tau(keys):
    rows = keys.shape[0]
    lo = jnp.zeros((rows, 1), jnp.int32)
    hi = jnp.full((rows, 1), _HI_BITS, jnp.int32)

    def body(_, carry):
        lo, hi = carry
        mid = (lo + hi) >> 1
        cnt = jnp.sum((keys <= mid).astype(jnp.int32), axis=1, keepdims=True)
        take_hi = cnt >= _RANK
        return (jnp.where(take_hi, lo, mid + 1), jnp.where(take_hi, mid, hi))

    lo, hi = jax.lax.fori_loop(0, 30, body, (lo, hi))
    s1_bits = lo

    cnt_leq = jnp.sum((keys <= s1_bits).astype(jnp.int32), axis=1, keepdims=True)
    above = jnp.where(keys > s1_bits, keys, _MAX_BITS)
    nxt_bits = jnp.min(above, axis=1, keepdims=True)
    s2_bits = jnp.where(cnt_leq >= _RANK + 1, s1_bits, nxt_bits)

    s1 = jax.lax.bitcast_convert_type(s1_bits, jnp.float32)
    s2 = jax.lax.bitcast_convert_type(s2_bits, jnp.float32)
    g = jnp.float32(_GAMMA)
    return s1 * (jnp.float32(1) - g) + s2 * g


def _masks_kernel(a_ref, b_ref, new_ref, dis_ref):
    a = a_ref[...]
    b = b_ref[...]
    ka = jax.lax.bitcast_convert_type(a, jnp.int32)
    kb = jax.lax.bitcast_convert_type(b, jnp.int32)
    rows = ka.shape[0]
    tau = _row_tau(jnp.concatenate([ka, kb], axis=0))
    tau_a = tau[:rows]
    tau_b = tau[rows:]
    a_hi = a > tau_a
    b_hi = b > tau_b
    new_ref[...] = a_hi & jnp.logical_not(b_hi)
    dis_ref[...] = b_hi & jnp.logical_not(a_hi)


@jax.jit
def kernel(C_now2past, C_past2now):
    rows, cols = C_now2past.shape
    block_rows = 8
    grid = (rows // block_rows,)
    spec = pl.BlockSpec((block_rows, cols), lambda i: (i, 0))
    out_shape = jax.ShapeDtypeStruct((rows, cols), jnp.bool_)
    new_mask, dis_mask = pl.pallas_call(
        _masks_kernel,
        grid=grid,
        in_specs=[spec, spec],
        out_specs=[spec, spec],
        out_shape=[out_shape, out_shape],
    )(C_now2past, C_past2now)
    return (new_mask, dis_mask)


# speedup vs baseline: 33.0050x; 1.2887x over previous
import functools

import jax
import jax.numpy as jnp
import numpy as np
from jax.experimental import pallas as pl

N_COLS = 32768
K_RATIO = 0.1

_POS = np.float32(1.0 - K_RATIO) * np.float32(N_COLS - 1)
_LO_IDX = int(np.floor(_POS))
_GAMMA = float(np.float32(_POS) - np.float32(_LO_IDX))
_RANK = _LO_IDX + 1

_HI_BITS = 0x3F800000
_MAX_BITS = 0x7F7FFFFF


def _row_tau(keys):
    rows = keys.shape[0]
    lo = jnp.zeros((rows, 1), jnp.int32)
    hi = jnp.full((rows, 1), _HI_BITS, jnp.int32)

    def body(_, carry):
        lo, hi = carry
        mid = (lo + hi) >> 1
        cnt = jnp.sum((keys <= mid).astype(jnp.int32), axis=1, keepdims=True)
        take_hi = cnt >= _RANK
        return (jnp.where(take_hi, lo, mid + 1), jnp.where(take_hi, mid, hi))

    lo, hi = jax.lax.fori_loop(0, 30, body, (lo, hi))
    s1_bits = lo

    cnt_leq = jnp.sum((keys <= s1_bits).astype(jnp.int32), axis=1, keepdims=True)
    above = jnp.where(keys > s1_bits, keys, _MAX_BITS)
    nxt_bits = jnp.min(above, axis=1, keepdims=True)
    s2_bits = jnp.where(cnt_leq >= _RANK + 1, s1_bits, nxt_bits)

    s1 = jax.lax.bitcast_convert_type(s1_bits, jnp.float32)
    s2 = jax.lax.bitcast_convert_type(s2_bits, jnp.float32)
    g = jnp.float32(_GAMMA)
    return s1 * (jnp.float32(1) - g) + s2 * g


def _masks_kernel(a_ref, b_ref, new_ref, dis_ref):
    a = a_ref[...]
    b = b_ref[...]
    ka = jax.lax.bitcast_convert_type(a, jnp.int32)
    kb = jax.lax.bitcast_convert_type(b, jnp.int32)
    rows = ka.shape[0]
    tau = _row_tau(jnp.concatenate([ka, kb], axis=0))
    tau_a = tau[:rows]
    tau_b = tau[rows:]
    a_hi = a > tau_a
    b_hi = b > tau_b
    new_ref[...] = a_hi & jnp.logical_not(b_hi)
    dis_ref[...] = b_hi & jnp.logical_not(a_hi)


@jax.jit
def kernel(C_now2past, C_past2now):
    rows, cols = C_now2past.shape
    block_rows = 32
    grid = (rows // block_rows,)
    spec = pl.BlockSpec((block_rows, cols), lambda i: (i, 0))
    out_shape = jax.ShapeDtypeStruct((rows, cols), jnp.bool_)
    new_mask, dis_mask = pl.pallas_call(
        _masks_kernel,
        grid=grid,
        in_specs=[spec, spec],
        out_specs=[spec, spec],
        out_shape=[out_shape, out_shape],
    )(C_now2past, C_past2now)
    return (new_mask, dis_mask)


# speedup vs baseline: 33.5665x; 1.0170x over previous
import functools

import jax
import jax.numpy as jnp
import numpy as np
from jax.experimental import pallas as pl

N_COLS = 32768
K_RATIO = 0.1

_POS = np.float32(1.0 - K_RATIO) * np.float32(N_COLS - 1)
_LO_IDX = int(np.floor(_POS))
_GAMMA = float(np.float32(_POS) - np.float32(_LO_IDX))
_RANK = _LO_IDX + 1

_HI_BITS = 0x3F800000
_MAX_BITS = 0x7F7FFFFF


def _row_tau(keys):
    rows = keys.shape[0]
    lo = jnp.zeros((rows, 1), jnp.int32)
    hi = jnp.full((rows, 1), _HI_BITS, jnp.int32)

    def body(_, carry):
        lo, hi = carry
        mid = (lo + hi) >> 1
        cnt = jnp.sum((keys <= mid).astype(jnp.int32), axis=1, keepdims=True)
        take_hi = cnt >= _RANK
        return (jnp.where(take_hi, lo, mid + 1), jnp.where(take_hi, mid, hi))

    lo, hi = jax.lax.fori_loop(0, 30, body, (lo, hi))
    s1_bits = lo

    cnt_leq = jnp.sum((keys <= s1_bits).astype(jnp.int32), axis=1, keepdims=True)
    above = jnp.where(keys > s1_bits, keys, _MAX_BITS)
    nxt_bits = jnp.min(above, axis=1, keepdims=True)
    s2_bits = jnp.where(cnt_leq >= _RANK + 1, s1_bits, nxt_bits)

    s1 = jax.lax.bitcast_convert_type(s1_bits, jnp.float32)
    s2 = jax.lax.bitcast_convert_type(s2_bits, jnp.float32)
    g = jnp.float32(_GAMMA)
    return s1 * (jnp.float32(1) - g) + s2 * g


def _masks_kernel(a_ref, b_ref, new_ref, dis_ref):
    a = a_ref[...]
    b = b_ref[...]
    ka = jax.lax.bitcast_convert_type(a, jnp.int32)
    kb = jax.lax.bitcast_convert_type(b, jnp.int32)
    rows = ka.shape[0]
    tau = _row_tau(jnp.concatenate([ka, kb], axis=0))
    tau_a = tau[:rows]
    tau_b = tau[rows:]
    a_hi = a > tau_a
    b_hi = b > tau_b
    new_ref[...] = a_hi & jnp.logical_not(b_hi)
    dis_ref[...] = b_hi & jnp.logical_not(a_hi)


@jax.jit
def kernel(C_now2past, C_past2now):
    rows, cols = C_now2past.shape
    block_rows = 64
    grid = (rows // block_rows,)
    spec = pl.BlockSpec((block_rows, cols), lambda i: (i, 0))
    out_shape = jax.ShapeDtypeStruct((rows, cols), jnp.bool_)
    new_mask, dis_mask = pl.pallas_call(
        _masks_kernel,
        grid=grid,
        in_specs=[spec, spec],
        out_specs=[spec, spec],
        out_shape=[out_shape, out_shape],
    )(C_now2past, C_past2now)
    return (new_mask, dis_mask)
